# Initial kernel scaffold; baseline (speedup 1.0000x reference)
import jax
import jax.numpy as jnp
from jax.experimental import pallas as pl


def kernel(user_inputs, item_inputs, user_embed, item_embed, group_embed, att_W1, att_b1, att_W2, att_b2, pred_W1, pred_b1, pred_W2, pred_b2, member_table, member_mask):
    raise NotImplementedError("write your pallas kernel here")


# speedup vs baseline: 28.1696x; 28.1696x over previous
import functools

import jax
import jax.numpy as jnp
from jax import lax
from jax.experimental import pallas as pl
from jax.experimental.pallas import tpu as pltpu
from jax.experimental.pallas import tpu_sc as plsc

_B = 16384
_D = 64
_G = 16
_M = 10
_LMD = 0.5
_CHUNK = 128
_MPAD = 256


def _sc_gather(item_embed, user_embed, item_idx, member_idx):
    info = plsc.get_sparse_core_info()
    nw = info.num_cores * info.num_subcores
    rows_per_w = _B // nw
    nchunk = rows_per_w // _CHUNK
    mesh = plsc.VectorSubcoreMesh(core_axis_name="c", subcore_axis_name="s")

    @functools.partial(
        pl.kernel,
        mesh=mesh,
        compiler_params=pltpu.CompilerParams(use_tc_tiling_on_sc=False),
        out_type=(
            jax.ShapeDtypeStruct((_B, _D), jnp.float32),
            jax.ShapeDtypeStruct((_MPAD, _D), jnp.float32),
        ),
        scratch_types=[
            pltpu.VMEM((nchunk, _CHUNK), jnp.int32),
            pltpu.VMEM((rows_per_w, _D), jnp.float32),
            pltpu.VMEM((2, _CHUNK), jnp.int32),
            pltpu.VMEM((_MPAD, _D), jnp.float32),
            pltpu.SemaphoreType.DMA,
        ],
    )
    def k(item_tbl, user_tbl, iidx_hbm, midx_hbm, out_items, out_members,
          iidx_v, irows_v, midx_v, mrows_v, sem):
        wid = lax.axis_index("s") * info.num_cores + lax.axis_index("c")
        base = wid * nchunk
        pltpu.sync_copy(iidx_hbm.at[pl.ds(base, nchunk)], iidx_v)
        cps = []
        for j in range(nchunk):
            cps.append(pltpu.async_copy(
                item_tbl.at[iidx_v.at[j]],
                irows_v.at[pl.ds(j * _CHUNK, _CHUNK)], sem))
        for cp in cps:
            cp.wait()
        pltpu.sync_copy(irows_v,
                        out_items.at[pl.ds(wid * rows_per_w, rows_per_w)])

        @pl.when(wid == 0)
        def _():
            pltpu.sync_copy(midx_hbm, midx_v)
            m0 = pltpu.async_copy(user_tbl.at[midx_v.at[0]],
                                  mrows_v.at[pl.ds(0, _CHUNK)], sem)
            m1 = pltpu.async_copy(user_tbl.at[midx_v.at[1]],
                                  mrows_v.at[pl.ds(_CHUNK, _CHUNK)], sem)
            m0.wait()
            m1.wait()
            pltpu.sync_copy(mrows_v, out_members)

    return k(item_embed, user_embed, item_idx, member_idx)


def _dot(a, b, dims):
    return lax.dot_general(a, b, (dims, ((), ())),
                           preferred_element_type=jnp.float32)


def _dense_body(item_ref, u_ref, mem_ref, mask_ref, ge_ref, w1m_ref, w1i_ref,
                b1_ref, w2_ref, b2_ref, pa_ref, pb_ref, pc_ref, pb1_ref,
                pw2_ref, pb2_ref, out_ref):
    f32 = jnp.float32
    mg = _M * _G
    item = item_ref[...]
    u = u_ref[...]
    gids = lax.broadcasted_iota(jnp.int32, (1, _G), 1)
    onehot = (u == gids).astype(f32)

    hi = _dot(item, w1i_ref[...], ((1,), (1,))) + b1_ref[...]

    mem = mem_ref[...]
    mem160 = mem[:mg, :]
    pre_all = _dot(mem160, w1m_ref[...], ((1,), (1,)))

    col = lax.broadcasted_iota(jnp.int32, (_G, mg), 1)
    row = lax.broadcasted_iota(jnp.int32, (_G, mg), 0)
    otile = (col % _G == row).astype(f32)
    expand = (col // _G == row).astype(f32)

    oht = _dot(onehot, otile, ((1,), (0,)))
    hit = _dot(hi, otile, ((1,), (0,)))
    pre_tiled = _dot(pre_all, otile, ((1,), (0,)))
    rr = lax.broadcasted_iota(jnp.int32, (mg, mg), 0)
    cc = lax.broadcasted_iota(jnp.int32, (mg, mg), 1)
    bd = (rr // _G == cc // _G).astype(f32)
    x = _dot(oht, pre_tiled * bd, ((1,), (0,))) + hit
    h_act = jnp.maximum(x, 0.0)

    w2t = _dot(w2_ref[...], otile, ((1,), (0,)))
    w2cat = expand * w2t
    logits = _dot(h_act, w2cat, ((1,), (1,))) + b2_ref[0, 0]
    logits = jnp.clip(logits, -50.0, 50.0)

    mq = _dot(onehot, mask_ref[...], ((1,), (0,)))
    wgt = jnp.exp(logits) * mq
    wgt = wgt / jnp.sum(wgt, axis=1, keepdims=True)

    woh = _dot(wgt, expand, ((1,), (0,))) * oht
    g_att = _dot(woh, mem160, ((1,), (0,)))
    ge = _dot(onehot, ge_ref[...], ((1,), (0,)))
    g = _LMD * g_att + ge

    gi = g * item
    hp = (_dot(gi, pa_ref[...], ((1,), (1,)))
          + _dot(g, pb_ref[...], ((1,), (1,)))
          + _dot(item, pc_ref[...], ((1,), (1,)))
          + pb1_ref[...])
    hp = jnp.maximum(hp, 0.0)
    preds8 = _dot(hp, pw2_ref[...], ((1,), (1,)))
    out_ref[...] = preds8[:, 0:1] + pb2_ref[0, 0]


def _dense_specs(block_b):
    full = lambda shape: pl.BlockSpec(shape, lambda i: (0, 0))
    smem = pl.BlockSpec(memory_space=pltpu.SMEM)
    in_specs = [
        pl.BlockSpec((block_b, _D), lambda i: (i, 0)),
        pl.BlockSpec((block_b, 1), lambda i: (i, 0)),
        full((_MPAD, _D)),
        full((_G, _G)),
        full((_G, _D)),
        full((16, _D)),
        full((16, _D)),
        full((1, 16)),
        full((1, 16)),
        smem,
        full((8, _D)),
        full((8, _D)),
        full((8, _D)),
        full((1, 8)),
        full((8, 8)),
        smem,
    ]
    out_specs = pl.BlockSpec((block_b, 1), lambda i: (i, 0))
    return in_specs, out_specs


def _dense_call(item_rows, u2d, member_rows, mask_pad, group_embed, w1m,
                w1i, b1_2d, att_w2, b2_2d, pw1a, pw1b, pw1c, pb1_2d, pw2,
                pb2_2d, block_b=2048):
    nblk = _B // block_b
    in_specs, out_specs = _dense_specs(block_b)
    return pl.pallas_call(
        _dense_body,
        grid=(nblk,),
        in_specs=in_specs,
        out_specs=out_specs,
        out_shape=jax.ShapeDtypeStruct((_B, 1), jnp.float32),
    )(item_rows, u2d, member_rows, mask_pad, group_embed, w1m, w1i, b1_2d,
      att_w2, b2_2d, pw1a, pw1b, pw1c, pb1_2d, pw2, pb2_2d)


def kernel(user_inputs, item_inputs, user_embed, item_embed, group_embed,
           att_W1, att_b1, att_W2, att_b2, pred_W1, pred_b1, pred_W2,
           pred_b2, member_table, member_mask):
    item_idx = item_inputs.reshape(_B // _CHUNK, _CHUNK)
    midx_flat = member_table.T.reshape(-1)
    midx = jnp.concatenate(
        [midx_flat,
         jnp.zeros((_MPAD - midx_flat.shape[0],), jnp.int32)]).reshape(
             2, _CHUNK)

    item_rows, member_rows = _sc_gather(item_embed, user_embed, item_idx,
                                        midx)

    u2d = user_inputs.reshape(_B, 1)
    mask_pad = jnp.concatenate(
        [member_mask, jnp.zeros((_G, _G - _M), jnp.float32)], axis=1)
    w1m = att_W1[:, :_D]
    w1i = att_W1[:, _D:]
    b1_2d = att_b1.reshape(1, 16)
    b2_2d = att_b2.reshape(1, 1)
    pw1a = pred_W1[:, :_D]
    pw1b = pred_W1[:, _D:2 * _D]
    pw1c = pred_W1[:, 2 * _D:]
    pb1_2d = pred_b1.reshape(1, 8)
    pw2pad = jnp.concatenate([pred_W2, jnp.zeros((7, 8), jnp.float32)],
                             axis=0)
    pb2_2d = pred_b2.reshape(1, 1)

    return _dense_call(item_rows, u2d, member_rows, mask_pad, group_embed,
                       w1m, w1i, b1_2d, att_W2, b2_2d, pw1a, pw1b, pw1c,
                       pb1_2d, pw2pad, pb2_2d)
